# baseline (speedup 1.0000x reference)
import functools

import jax
import jax.numpy as jnp
import numpy as np
from jax import lax
from jax.experimental import pallas as pl
from jax.experimental.pallas import tpu as pltpu
from jax.experimental.pallas import tpu_sc as plsc

N = 10000
E = 160000
H = 128
EF = 16
NPAD = 10000
NGRID = 10
NBLK = NPAD // NGRID
EB = 1280
EGRID = E // EB
B = 40
NWIN = E // B
NSUB = 16
ACC_ROWS = 10112
ROWS_PER_TILE = ACC_ROWS // NSUB
INV_SQRT_3 = 1.0 / float(np.sqrt(3.0))
INV_SQRT_H = 1.0 / float(np.sqrt(H))


def _node_tables_body(x_ref, xd_in_ref, vec_ref, Wd_ref, bd_ref, W1_ref, b1_ref,
                      W2_ref, b2_ref, tj_ref, ti_ref):
    xd = xd_in_ref[...] @ Wd_ref[...] + bd_ref[...][None, :]
    h = x_ref[...] @ W1_ref[...] + b1_ref[...][None, :]
    h = h * jax.nn.sigmoid(h) * (1.0 / 0.6)
    xh = h @ W2_ref[...] + b2_ref[...][None, :]
    for c in range(4):
        for p in range(3):
            col = 128 * p + 32 * c
            tj_ref[c, :, 32 * p:32 * p + 32] = xh[:, col:col + 32]
            tj_ref[c, :, 96 + 32 * p:96 + 32 * p + 32] = xd[:, col:col + 32]
            tj_ref[c, :, 192 + 32 * p:192 + 32 * p + 32] = (
                vec_ref[:, p, 32 * c:32 * c + 32] * INV_SQRT_H)
            ti_ref[c, :, 32 * p:32 * p + 32] = xd[:, col:col + 32]


def _rbf_body(ef_ref, We_ref, be_ref, j_ref, i_ref, evec_ref,
              out_ref, jj_ref, ii_ref, ev_ref):
    r = (ef_ref[...] @ We_ref[...] + be_ref[...][None, :]) * INV_SQRT_3
    for c in range(4):
        for p in range(3):
            col = 128 * p + 32 * c
            out_ref[c, :, 32 * p:32 * p + 32] = r[:, col:col + 32]
        jj_ref[c, :] = j_ref[0, 0, :] + c * NPAD
        ii_ref[c, :] = i_ref[0, 0, :] + c * NPAD
    ones16 = jnp.ones((1, 16), jnp.float32)
    for p in range(3):
        ev_ref[:, 16 * p:16 * p + 16] = (
            evec_ref[:, p:p + 1] * INV_SQRT_H) * ones16


def _sc_edge_body(tj_hbm, ti_hbm, rbf_hbm, jj_hbm, ii_hbm, i_hbm, ev_hbm, out_hbm,
                  i_v, jj_v, ii_v, ev_v, rbf_v, tj_v, ti_v, upd_v, zbuf,
                  acc, sem1, sem2, *lsems):
    core = lax.axis_index("c")
    sid = lax.axis_index("s")
    nwin = NWIN // NSUB

    def _load_args(lw, b, chunk):
        e0 = (sid + lw * NSUB) * B
        return ((jj_hbm.at[chunk, pl.ds(e0, B)], jj_v.at[b], lsems[b * 5 + 0]),
                (ii_hbm.at[chunk, pl.ds(e0, B)], ii_v.at[b], lsems[b * 5 + 1]),
                (i_hbm.at[pl.ds(e0, B)], i_v.at[b], lsems[b * 5 + 2]),
                (ev_hbm.at[pl.ds(e0, B)], ev_v.at[b], lsems[b * 5 + 3]),
                (rbf_hbm.at[chunk, pl.ds(e0, B)], rbf_v.at[b], lsems[b * 5 + 4]))

    def _issue_loads(lw, b, chunk):
        for src, dst, sem in _load_args(lw, b, chunk):
            pltpu.async_copy(src, dst, sem)

    def _wait_loads(lw, b, chunk):
        for src, dst, sem in _load_args(lw, b, chunk):
            pltpu.make_async_copy(src, dst, sem).wait()

    def _zrow(r, _):
        for k in range(8):
            zbuf[r, pl.ds(k * 16, 16)] = jnp.zeros((16,), jnp.float32)
        return 0
    lax.fori_loop(0, 8, _zrow, 0)

    def _make_edge_body(b):
        def _edge_quad(q, _):
            for r in range(4):
                e = q * 4 + r
                mregs = []
                for k in range(6):
                    xh = tj_v[e, pl.ds(k * 16, 16)]
                    xdj = tj_v[e, pl.ds(96 + k * 16, 16)]
                    xdi = ti_v[e, pl.ds(k * 16, 16)]
                    rb = rbf_v[b, e, pl.ds(k * 16, 16)]
                    mregs.append(xh * (xdj + xdi) * rb)
                for p in range(3):
                    evp = ev_v[b, e, pl.ds(p * 16, 16)]
                    for k in range(2):
                        v = tj_v[e, pl.ds(192 + p * 32 + k * 16, 16)]
                        upd_v[e, pl.ds(p * 32 + k * 16, 16)] = (
                            mregs[k] * v + mregs[2 + k] * evp)
                upd_v[e, pl.ds(96, 16)] = mregs[4]
                upd_v[e, pl.ds(112, 16)] = mregs[5]
            return 0
        return _edge_quad

    for t in range(2):
        chunk = core * 2 + t

        def _zacc(kk, _):
            pltpu.sync_copy(zbuf, acc.at[pl.ds(sid * ROWS_PER_TILE + kk * 8, 8)])
            return 0
        lax.fori_loop(0, ROWS_PER_TILE // 8, _zacc, 0)
        plsc.subcore_barrier()

        _issue_loads(0, 0, chunk)
        _issue_loads(1, 1, chunk)

        def _pair_body(g, _):
            for b in range(2):
                lw = 2 * g + b
                _wait_loads(lw, b, chunk)
                gj = pltpu.async_copy(tj_hbm.at[jj_v.at[b]], tj_v, sem1)
                gi = pltpu.async_copy(ti_hbm.at[ii_v.at[b]], ti_v, sem2)
                gj.wait()
                gi.wait()
                lax.fori_loop(0, B // 4, _make_edge_body(b), 0)
                pltpu.sync_copy(upd_v, acc.at[i_v.at[b]], add=True)
                _issue_loads(jnp.minimum(lw + 2, nwin - 1), b, chunk)
            return 0

        lax.fori_loop(0, nwin // 2, _pair_body, 0)
        _wait_loads(nwin - 1, 0, chunk)
        _wait_loads(nwin - 1, 1, chunk)
        plsc.subcore_barrier()

        pltpu.sync_copy(acc.at[pl.ds(sid * ROWS_PER_TILE, ROWS_PER_TILE)],
                        out_hbm.at[chunk, pl.ds(sid * ROWS_PER_TILE, ROWS_PER_TILE)])
        plsc.subcore_barrier()


@functools.lru_cache(maxsize=1)
def _get_sc_edge():
    return functools.partial(
        pl.kernel,
        out_type=jax.ShapeDtypeStruct((4, ACC_ROWS, 128), jnp.float32),
        mesh=plsc.VectorSubcoreMesh(core_axis_name="c", subcore_axis_name="s"),
        compiler_params=pltpu.CompilerParams(use_tc_tiling_on_sc=False),
        scratch_types=[
            pltpu.VMEM((2, B), jnp.int32),
            pltpu.VMEM((2, B), jnp.int32),
            pltpu.VMEM((2, B), jnp.int32),
            pltpu.VMEM((2, B, 48), jnp.float32),
            pltpu.VMEM((2, B, 96), jnp.float32),
            pltpu.VMEM((B, 384), jnp.float32),
            pltpu.VMEM((B, 128), jnp.float32),
            pltpu.VMEM((B, 128), jnp.float32),
            pltpu.VMEM((8, 128), jnp.float32),
            pltpu.VMEM_SHARED((ACC_ROWS, 128), jnp.float32),
        ] + [pltpu.SemaphoreType.DMA] * 12,
    )(_sc_edge_body)


def _act_body(acc_ref, Wv_ref, Wg_ref, dx_ref, dvec_ref):
    nb = acc_ref.shape[1]
    dv_parts = []
    for p in range(3):
        cols = [acc_ref[c, :, 32 * p:32 * p + 32] for c in range(4)]
        dv_parts.append(jnp.concatenate(cols, axis=1))
    dv = jnp.stack(dv_parts, axis=1)
    for c in range(4):
        dx_ref[:, 32 * c:32 * c + 32] = acc_ref[c, :, 96:128]
    dv2 = dv.reshape(nb * 3, H)
    gvec = (dv2 @ Wg_ref[...]).reshape(nb, 3, 1)
    lvec = (dv2 @ Wv_ref[...]).reshape(nb, 3, H)
    dot = jnp.sum(lvec * gvec, axis=1, keepdims=True)
    mask = (dot >= 0).astype(jnp.float32)
    dvec_ref[...] = mask * lvec + (1.0 - mask) * (lvec + gvec) * 0.5


def kernel(x, x_defect, vec, edge_index, edge_feat, edge_vector,
           We, be, Wd, bd, W1, b1, W2, b2, Wv, Wg):
    jarr = edge_index[0].astype(jnp.int32)
    iarr = edge_index[1].astype(jnp.int32)

    tj, ti = pl.pallas_call(
        _node_tables_body,
        grid=(NGRID,),
        in_specs=[pl.BlockSpec((NBLK, H), lambda g: (g, 0)),
                  pl.BlockSpec((NBLK, H), lambda g: (g, 0)),
                  pl.BlockSpec((NBLK, 3, H), lambda g: (g, 0, 0)),
                  pl.BlockSpec((H, 3 * H), lambda g: (0, 0)),
                  pl.BlockSpec((3 * H,), lambda g: (0,)),
                  pl.BlockSpec((H, H // 2), lambda g: (0, 0)),
                  pl.BlockSpec((H // 2,), lambda g: (0,)),
                  pl.BlockSpec((H // 2, 3 * H), lambda g: (0, 0)),
                  pl.BlockSpec((3 * H,), lambda g: (0,))],
        out_specs=(pl.BlockSpec((4, NBLK, 384), lambda g: (0, g, 0)),
                   pl.BlockSpec((4, NBLK, 128), lambda g: (0, g, 0))),
        out_shape=(jax.ShapeDtypeStruct((4, NPAD, 384), jnp.float32),
                   jax.ShapeDtypeStruct((4, NPAD, 128), jnp.float32)),
    )(x, x_defect, vec, Wd, bd, W1, b1, W2, b2)

    rbf, jj, ii, ev = pl.pallas_call(
        _rbf_body,
        grid=(EGRID,),
        in_specs=[pl.BlockSpec((EB, EF), lambda g: (g, 0)),
                  pl.BlockSpec((EF, 3 * H), lambda g: (0, 0)),
                  pl.BlockSpec((3 * H,), lambda g: (0,)),
                  pl.BlockSpec((1, 1, EB), lambda g: (g, 0, 0)),
                  pl.BlockSpec((1, 1, EB), lambda g: (g, 0, 0)),
                  pl.BlockSpec((EB, 3), lambda g: (g, 0))],
        out_specs=(pl.BlockSpec((4, EB, 96), lambda g: (0, g, 0)),
                   pl.BlockSpec((4, EB), lambda g: (0, g)),
                   pl.BlockSpec((4, EB), lambda g: (0, g)),
                   pl.BlockSpec((EB, 48), lambda g: (g, 0))),
        out_shape=(jax.ShapeDtypeStruct((4, E, 96), jnp.float32),
                   jax.ShapeDtypeStruct((4, E), jnp.int32),
                   jax.ShapeDtypeStruct((4, E), jnp.int32),
                   jax.ShapeDtypeStruct((E, 48), jnp.float32)),
    )(edge_feat, We, be, jarr.reshape(EGRID, 1, EB), iarr.reshape(EGRID, 1, EB),
      edge_vector.astype(jnp.float32))

    tj2 = tj.reshape(4 * NPAD, 384)
    ti2 = ti.reshape(4 * NPAD, 128)

    out_acc = _get_sc_edge()(tj2, ti2, rbf, jj, ii, iarr, ev)

    d_x, d_vec = pl.pallas_call(
        _act_body,
        grid=(NGRID,),
        in_specs=[pl.BlockSpec((4, NBLK, 128), lambda g: (0, g, 0)),
                  pl.BlockSpec((H, H), lambda g: (0, 0)),
                  pl.BlockSpec((H, 1), lambda g: (0, 0))],
        out_specs=(pl.BlockSpec((NBLK, H), lambda g: (g, 0)),
                   pl.BlockSpec((NBLK, 3, H), lambda g: (g, 0, 0))),
        out_shape=(jax.ShapeDtypeStruct((N, H), jnp.float32),
                   jax.ShapeDtypeStruct((N, 3, H), jnp.float32)),
    )(out_acc, Wv, Wg)

    return (d_x, d_vec)


# speedup vs baseline: 1.0134x; 1.0134x over previous
import functools

import jax
import jax.numpy as jnp
import numpy as np
from jax import lax
from jax.experimental import pallas as pl
from jax.experimental.pallas import tpu as pltpu
from jax.experimental.pallas import tpu_sc as plsc

N = 10000
E = 160000
H = 128
EF = 16
NPAD = 10240
NGRID = 10
NBLK = NPAD // NGRID
EB = 1280
EGRID = E // EB
B = 40
NWIN = E // B
NSUB = 16
ACC_ROWS = 10112
ROWS_PER_TILE = ACC_ROWS // NSUB
INV_SQRT_3 = 1.0 / float(np.sqrt(3.0))
INV_SQRT_H = 1.0 / float(np.sqrt(H))


def _node_tables_body(x_ref, xd_in_ref, vec_ref, Wd_ref, bd_ref, W1_ref, b1_ref,
                      W2_ref, b2_ref, tj_ref, ti_ref):
    xd = xd_in_ref[...] @ Wd_ref[...] + bd_ref[...][None, :]
    h = x_ref[...] @ W1_ref[...] + b1_ref[...][None, :]
    h = h * jax.nn.sigmoid(h) * (1.0 / 0.6)
    xh = h @ W2_ref[...] + b2_ref[...][None, :]
    for c in range(4):
        for p in range(3):
            col = 128 * p + 32 * c
            tj_ref[c, :, 32 * p:32 * p + 32] = xh[:, col:col + 32]
            tj_ref[c, :, 96 + 32 * p:96 + 32 * p + 32] = xd[:, col:col + 32]
            tj_ref[c, :, 192 + 32 * p:192 + 32 * p + 32] = (
                vec_ref[:, p, 32 * c:32 * c + 32] * INV_SQRT_H)
            ti_ref[c, :, 32 * p:32 * p + 32] = xd[:, col:col + 32]


def _rbf_body(ef_ref, We_ref, be_ref, j_ref, i_ref, out_ref, jj_ref, ii_ref):
    r = (ef_ref[...] @ We_ref[...] + be_ref[...][None, :]) * INV_SQRT_3
    for c in range(4):
        for p in range(3):
            col = 128 * p + 32 * c
            out_ref[c, :, 32 * p:32 * p + 32] = r[:, col:col + 32]
        jj_ref[c, :] = j_ref[0, 0, :] + c * NPAD
        ii_ref[c, :] = i_ref[0, 0, :] + c * NPAD


def _sc_edge_body(tj_hbm, ti_hbm, rbf_hbm, jj_hbm, ii_hbm, i_hbm, ev_hbm, out_hbm,
                  i_v, jj_v, ii_v, ev_v, rbf_v, tj_v, ti_v, upd_v, zbuf,
                  acc, sem1, sem2, *lsems):
    core = lax.axis_index("c")
    sid = lax.axis_index("s")
    nwin = NWIN // NSUB

    def _load_args(lw, b, chunk):
        e0 = (sid + lw * NSUB) * B
        return ((jj_hbm.at[chunk, pl.ds(e0, B)], jj_v.at[b], lsems[b * 5 + 0]),
                (ii_hbm.at[chunk, pl.ds(e0, B)], ii_v.at[b], lsems[b * 5 + 1]),
                (i_hbm.at[pl.ds(e0, B)], i_v.at[b], lsems[b * 5 + 2]),
                (ev_hbm.at[pl.ds(e0, B)], ev_v.at[b], lsems[b * 5 + 3]),
                (rbf_hbm.at[chunk, pl.ds(e0, B)], rbf_v.at[b], lsems[b * 5 + 4]))

    def _issue_loads(lw, b, chunk):
        for src, dst, sem in _load_args(lw, b, chunk):
            pltpu.async_copy(src, dst, sem)

    def _wait_loads(lw, b, chunk):
        for src, dst, sem in _load_args(lw, b, chunk):
            pltpu.make_async_copy(src, dst, sem).wait()

    def _zrow(r, _):
        for k in range(8):
            zbuf[r, pl.ds(k * 16, 16)] = jnp.zeros((16,), jnp.float32)
        return 0
    lax.fori_loop(0, 8, _zrow, 0)

    def _make_edge_body(b):
        def _edge_body(e, _):
            mregs = []
            for k in range(6):
                xh = tj_v[e, pl.ds(k * 16, 16)]
                xdj = tj_v[e, pl.ds(96 + k * 16, 16)]
                xdi = ti_v[e, pl.ds(k * 16, 16)]
                rb = rbf_v[b, e, pl.ds(k * 16, 16)]
                mregs.append(xh * (xdj + xdi) * rb)
            for p in range(3):
                evp = ev_v[b, e, pl.ds(p * 16, 16)]
                for k in range(2):
                    v = tj_v[e, pl.ds(192 + p * 32 + k * 16, 16)]
                    upd_v[e, pl.ds(p * 32 + k * 16, 16)] = (
                        mregs[k] * v + mregs[2 + k] * evp)
            upd_v[e, pl.ds(96, 16)] = mregs[4]
            upd_v[e, pl.ds(112, 16)] = mregs[5]
            return 0
        return _edge_body

    for t in range(2):
        chunk = core * 2 + t

        def _zacc(kk, _):
            pltpu.sync_copy(zbuf, acc.at[pl.ds(sid * ROWS_PER_TILE + kk * 8, 8)])
            return 0
        lax.fori_loop(0, ROWS_PER_TILE // 8, _zacc, 0)
        plsc.subcore_barrier()

        _issue_loads(0, 0, chunk)
        _issue_loads(1, 1, chunk)

        def _pair_body(g, _):
            for b in range(2):
                lw = 2 * g + b
                _wait_loads(lw, b, chunk)
                gj = pltpu.async_copy(tj_hbm.at[jj_v.at[b]], tj_v, sem1)
                gi = pltpu.async_copy(ti_hbm.at[ii_v.at[b]], ti_v, sem2)
                gj.wait()
                gi.wait()
                lax.fori_loop(0, B, _make_edge_body(b), 0)
                pltpu.sync_copy(upd_v, acc.at[i_v.at[b]], add=True)
                _issue_loads(jnp.minimum(lw + 2, nwin - 1), b, chunk)
            return 0

        lax.fori_loop(0, nwin // 2, _pair_body, 0)
        _wait_loads(nwin - 1, 0, chunk)
        _wait_loads(nwin - 1, 1, chunk)
        plsc.subcore_barrier()

        pltpu.sync_copy(acc.at[pl.ds(sid * ROWS_PER_TILE, ROWS_PER_TILE)],
                        out_hbm.at[chunk, pl.ds(sid * ROWS_PER_TILE, ROWS_PER_TILE)])
        plsc.subcore_barrier()


@functools.lru_cache(maxsize=1)
def _get_sc_edge():
    return functools.partial(
        pl.kernel,
        out_type=jax.ShapeDtypeStruct((4, NPAD, 128), jnp.float32),
        mesh=plsc.VectorSubcoreMesh(core_axis_name="c", subcore_axis_name="s"),
        compiler_params=pltpu.CompilerParams(use_tc_tiling_on_sc=False),
        scratch_types=[
            pltpu.VMEM((2, B), jnp.int32),
            pltpu.VMEM((2, B), jnp.int32),
            pltpu.VMEM((2, B), jnp.int32),
            pltpu.VMEM((2, B, 48), jnp.float32),
            pltpu.VMEM((2, B, 96), jnp.float32),
            pltpu.VMEM((B, 384), jnp.float32),
            pltpu.VMEM((B, 128), jnp.float32),
            pltpu.VMEM((B, 128), jnp.float32),
            pltpu.VMEM((8, 128), jnp.float32),
            pltpu.VMEM_SHARED((ACC_ROWS, 128), jnp.float32),
        ] + [pltpu.SemaphoreType.DMA] * 12,
    )(_sc_edge_body)


def _act_body(acc_ref, Wv_ref, Wg_ref, dx_ref, dvec_ref):
    nb = acc_ref.shape[1]
    dv_parts = []
    for p in range(3):
        cols = [acc_ref[c, :, 32 * p:32 * p + 32] for c in range(4)]
        dv_parts.append(jnp.concatenate(cols, axis=1))
    dv = jnp.stack(dv_parts, axis=1)
    for c in range(4):
        dx_ref[:, 32 * c:32 * c + 32] = acc_ref[c, :, 96:128]
    dv2 = dv.reshape(nb * 3, H)
    gvec = (dv2 @ Wg_ref[...]).reshape(nb, 3, 1)
    lvec = (dv2 @ Wv_ref[...]).reshape(nb, 3, H)
    dot = jnp.sum(lvec * gvec, axis=1, keepdims=True)
    mask = (dot >= 0).astype(jnp.float32)
    dvec_ref[...] = mask * lvec + (1.0 - mask) * (lvec + gvec) * 0.5


def kernel(x, x_defect, vec, edge_index, edge_feat, edge_vector,
           We, be, Wd, bd, W1, b1, W2, b2, Wv, Wg):
    jarr = edge_index[0].astype(jnp.int32)
    iarr = edge_index[1].astype(jnp.int32)

    xp = jnp.zeros((NPAD, H), jnp.float32).at[:N].set(x)
    xdp = jnp.zeros((NPAD, H), jnp.float32).at[:N].set(x_defect)
    vecp = jnp.zeros((NPAD, 3, H), jnp.float32).at[:N].set(vec)

    tj, ti = pl.pallas_call(
        _node_tables_body,
        grid=(NGRID,),
        in_specs=[pl.BlockSpec((NBLK, H), lambda g: (g, 0)),
                  pl.BlockSpec((NBLK, H), lambda g: (g, 0)),
                  pl.BlockSpec((NBLK, 3, H), lambda g: (g, 0, 0)),
                  pl.BlockSpec((H, 3 * H), lambda g: (0, 0)),
                  pl.BlockSpec((3 * H,), lambda g: (0,)),
                  pl.BlockSpec((H, H // 2), lambda g: (0, 0)),
                  pl.BlockSpec((H // 2,), lambda g: (0,)),
                  pl.BlockSpec((H // 2, 3 * H), lambda g: (0, 0)),
                  pl.BlockSpec((3 * H,), lambda g: (0,))],
        out_specs=(pl.BlockSpec((4, NBLK, 384), lambda g: (0, g, 0)),
                   pl.BlockSpec((4, NBLK, 128), lambda g: (0, g, 0))),
        out_shape=(jax.ShapeDtypeStruct((4, NPAD, 384), jnp.float32),
                   jax.ShapeDtypeStruct((4, NPAD, 128), jnp.float32)),
    )(xp, xdp, vecp, Wd, bd, W1, b1, W2, b2)

    rbf, jj, ii = pl.pallas_call(
        _rbf_body,
        grid=(EGRID,),
        in_specs=[pl.BlockSpec((EB, EF), lambda g: (g, 0)),
                  pl.BlockSpec((EF, 3 * H), lambda g: (0, 0)),
                  pl.BlockSpec((3 * H,), lambda g: (0,)),
                  pl.BlockSpec((1, 1, EB), lambda g: (g, 0, 0)),
                  pl.BlockSpec((1, 1, EB), lambda g: (g, 0, 0))],
        out_specs=(pl.BlockSpec((4, EB, 96), lambda g: (0, g, 0)),
                   pl.BlockSpec((4, EB), lambda g: (0, g)),
                   pl.BlockSpec((4, EB), lambda g: (0, g))),
        out_shape=(jax.ShapeDtypeStruct((4, E, 96), jnp.float32),
                   jax.ShapeDtypeStruct((4, E), jnp.int32),
                   jax.ShapeDtypeStruct((4, E), jnp.int32)),
    )(edge_feat, We, be, jarr.reshape(EGRID, 1, EB), iarr.reshape(EGRID, 1, EB))

    tj2 = tj.reshape(4 * NPAD, 384)
    ti2 = ti.reshape(4 * NPAD, 128)
    ev = jnp.repeat(edge_vector.astype(jnp.float32) * INV_SQRT_H, 16, axis=1)

    out_acc = _get_sc_edge()(tj2, ti2, rbf, jj, ii, iarr, ev)

    d_x, d_vec = pl.pallas_call(
        _act_body,
        grid=(NGRID,),
        in_specs=[pl.BlockSpec((4, NBLK, 128), lambda g: (0, g, 0)),
                  pl.BlockSpec((H, H), lambda g: (0, 0)),
                  pl.BlockSpec((H, 1), lambda g: (0, 0))],
        out_specs=(pl.BlockSpec((NBLK, H), lambda g: (g, 0)),
                   pl.BlockSpec((NBLK, 3, H), lambda g: (g, 0, 0))),
        out_shape=(jax.ShapeDtypeStruct((NPAD, H), jnp.float32),
                   jax.ShapeDtypeStruct((NPAD, 3, H), jnp.float32)),
    )(out_acc, Wv, Wg)

    return (d_x[:N], d_vec[:N])


# speedup vs baseline: 1.0249x; 1.0114x over previous
import functools

import jax
import jax.numpy as jnp
import numpy as np
from jax import lax
from jax.experimental import pallas as pl
from jax.experimental.pallas import tpu as pltpu
from jax.experimental.pallas import tpu_sc as plsc

N = 10000
E = 160000
H = 128
EF = 16
NPAD = 10240
NGRID = 10
NBLK = NPAD // NGRID
EPAD = 163840
EB = 1280
EGRID = EPAD // EB
B = 32
NWIN = EPAD // B
NSUB = 16
ACC_ROWS = 10112
ROWS_PER_TILE = ACC_ROWS // NSUB
INV_SQRT_3 = 1.0 / float(np.sqrt(3.0))
INV_SQRT_H = 1.0 / float(np.sqrt(H))


def _node_tables_body(x_ref, xd_in_ref, vec_ref, Wd_ref, bd_ref, W1_ref, b1_ref,
                      W2_ref, b2_ref, tj_ref, ti_ref):
    xd = xd_in_ref[...] @ Wd_ref[...] + bd_ref[...][None, :]
    h = x_ref[...] @ W1_ref[...] + b1_ref[...][None, :]
    h = h * jax.nn.sigmoid(h) * (1.0 / 0.6)
    xh = h @ W2_ref[...] + b2_ref[...][None, :]
    for c in range(4):
        for p in range(3):
            col = 128 * p + 32 * c
            tj_ref[c, :, 32 * p:32 * p + 32] = xh[:, col:col + 32]
            tj_ref[c, :, 96 + 32 * p:96 + 32 * p + 32] = xd[:, col:col + 32]
            tj_ref[c, :, 192 + 32 * p:192 + 32 * p + 32] = (
                vec_ref[:, p, 32 * c:32 * c + 32] * INV_SQRT_H)
            ti_ref[c, :, 32 * p:32 * p + 32] = xd[:, col:col + 32]


def _rbf_body(ef_ref, We_ref, be_ref, j_ref, i_ref, out_ref, jj_ref, ii_ref):
    r = (ef_ref[...] @ We_ref[...] + be_ref[...][None, :]) * INV_SQRT_3
    for c in range(4):
        for p in range(3):
            col = 128 * p + 32 * c
            out_ref[c, :, 32 * p:32 * p + 32] = r[:, col:col + 32]
        jj_ref[c, :] = j_ref[0, 0, :] + c * NPAD
        ii_ref[c, :] = i_ref[0, 0, :] + c * NPAD


def _sc_edge_body(tj_hbm, ti_hbm, rbf_hbm, jj_hbm, ii_hbm, i_hbm, ev_hbm, out_hbm,
                  i_v, jj_v, ii_v, ev_v, rbf_v, tj_v, ti_v, upd_v, zbuf,
                  acc, *sems):
    core = lax.axis_index("c")
    sid = lax.axis_index("s")
    nwin = NWIN // NSUB

    def _load_args(lw, b, chunk):
        e0 = (sid + lw * NSUB) * B
        return ((jj_hbm.at[chunk, pl.ds(e0, B)], jj_v.at[b], sems[4 + b * 5 + 0]),
                (ii_hbm.at[chunk, pl.ds(e0, B)], ii_v.at[b], sems[4 + b * 5 + 1]),
                (i_hbm.at[pl.ds(e0, B)], i_v.at[b], sems[4 + b * 5 + 2]),
                (ev_hbm.at[pl.ds(e0, B)], ev_v.at[b], sems[4 + b * 5 + 3]),
                (rbf_hbm.at[chunk, pl.ds(e0, B)], rbf_v.at[b], sems[4 + b * 5 + 4]))

    def _issue_loads(lw, b, chunk):
        for src, dst, sem in _load_args(lw, b, chunk):
            pltpu.async_copy(src, dst, sem)

    def _wait_loads(lw, b, chunk):
        for src, dst, sem in _load_args(lw, b, chunk):
            pltpu.make_async_copy(src, dst, sem).wait()

    def _gather_args(b):
        return ((tj_hbm.at[jj_v.at[b]], tj_v.at[b], sems[b * 2 + 0]),
                (ti_hbm.at[ii_v.at[b]], ti_v.at[b], sems[b * 2 + 1]))

    def _issue_gathers(b):
        for src, dst, sem in _gather_args(b):
            pltpu.async_copy(src, dst, sem)

    def _wait_gathers(b):
        for src, dst, sem in _gather_args(b):
            pltpu.make_async_copy(src, dst, sem).wait()

    def _zrow(r, _):
        for k in range(8):
            zbuf[r, pl.ds(k * 16, 16)] = jnp.zeros((16,), jnp.float32)
        return 0
    lax.fori_loop(0, 8, _zrow, 0)

    def _make_edge_body(b):
        def _edge_body(e, _):
            mregs = []
            for k in range(6):
                xh = tj_v[b, e, pl.ds(k * 16, 16)]
                xdj = tj_v[b, e, pl.ds(96 + k * 16, 16)]
                xdi = ti_v[b, e, pl.ds(k * 16, 16)]
                rb = rbf_v[b, e, pl.ds(k * 16, 16)]
                mregs.append(xh * (xdj + xdi) * rb)
            for p in range(3):
                evp = ev_v[b, e, pl.ds(p * 16, 16)]
                for k in range(2):
                    v = tj_v[b, e, pl.ds(192 + p * 32 + k * 16, 16)]
                    upd_v[e, pl.ds(p * 32 + k * 16, 16)] = (
                        mregs[k] * v + mregs[2 + k] * evp)
            upd_v[e, pl.ds(96, 16)] = mregs[4]
            upd_v[e, pl.ds(112, 16)] = mregs[5]
            return 0
        return _edge_body

    for t in range(2):
        chunk = core * 2 + t

        def _zacc(kk, _):
            pltpu.sync_copy(zbuf, acc.at[pl.ds(sid * ROWS_PER_TILE + kk * 8, 8)])
            return 0
        lax.fori_loop(0, ROWS_PER_TILE // 8, _zacc, 0)
        plsc.subcore_barrier()

        _issue_loads(0, 0, chunk)
        _issue_loads(1, 1, chunk)
        _wait_loads(0, 0, chunk)
        _issue_gathers(0)

        def _pair_body(g, _):
            for b in range(2):
                o = 1 - b
                lw = 2 * g + b
                _wait_gathers(b)
                _wait_loads(jnp.minimum(lw + 1, nwin - 1), o, chunk)
                _issue_gathers(o)
                lax.fori_loop(0, B, _make_edge_body(b), 0)
                pltpu.sync_copy(upd_v, acc.at[i_v.at[b]], add=True)
                _issue_loads(jnp.minimum(lw + 2, nwin - 1), b, chunk)
            return 0

        lax.fori_loop(0, nwin // 2, _pair_body, 0)
        _wait_gathers(0)
        _wait_loads(nwin - 1, 1, chunk)
        plsc.subcore_barrier()

        pltpu.sync_copy(acc.at[pl.ds(sid * ROWS_PER_TILE, ROWS_PER_TILE)],
                        out_hbm.at[chunk, pl.ds(sid * ROWS_PER_TILE, ROWS_PER_TILE)])
        plsc.subcore_barrier()


@functools.lru_cache(maxsize=1)
def _get_sc_edge():
    return functools.partial(
        pl.kernel,
        out_type=jax.ShapeDtypeStruct((4, NPAD, 128), jnp.float32),
        mesh=plsc.VectorSubcoreMesh(core_axis_name="c", subcore_axis_name="s"),
        compiler_params=pltpu.CompilerParams(use_tc_tiling_on_sc=False),
        scratch_types=[
            pltpu.VMEM((2, B), jnp.int32),
            pltpu.VMEM((2, B), jnp.int32),
            pltpu.VMEM((2, B), jnp.int32),
            pltpu.VMEM((2, B, 48), jnp.float32),
            pltpu.VMEM((2, B, 96), jnp.float32),
            pltpu.VMEM((2, B, 384), jnp.float32),
            pltpu.VMEM((2, B, 128), jnp.float32),
            pltpu.VMEM((B, 128), jnp.float32),
            pltpu.VMEM((8, 128), jnp.float32),
            pltpu.VMEM_SHARED((ACC_ROWS, 128), jnp.float32),
        ] + [pltpu.SemaphoreType.DMA] * 14,
    )(_sc_edge_body)


def _act_body(acc_ref, Wv_ref, Wg_ref, dx_ref, dvec_ref):
    nb = acc_ref.shape[1]
    dv_parts = []
    for p in range(3):
        cols = [acc_ref[c, :, 32 * p:32 * p + 32] for c in range(4)]
        dv_parts.append(jnp.concatenate(cols, axis=1))
    dv = jnp.stack(dv_parts, axis=1)
    for c in range(4):
        dx_ref[:, 32 * c:32 * c + 32] = acc_ref[c, :, 96:128]
    dv2 = dv.reshape(nb * 3, H)
    gvec = (dv2 @ Wg_ref[...]).reshape(nb, 3, 1)
    lvec = (dv2 @ Wv_ref[...]).reshape(nb, 3, H)
    dot = jnp.sum(lvec * gvec, axis=1, keepdims=True)
    mask = (dot >= 0).astype(jnp.float32)
    dvec_ref[...] = mask * lvec + (1.0 - mask) * (lvec + gvec) * 0.5


def kernel(x, x_defect, vec, edge_index, edge_feat, edge_vector,
           We, be, Wd, bd, W1, b1, W2, b2, Wv, Wg):
    jarr = jnp.zeros((EPAD,), jnp.int32).at[:E].set(edge_index[0].astype(jnp.int32))
    iarr = jnp.concatenate([
        edge_index[1].astype(jnp.int32),
        (jnp.arange(EPAD - E, dtype=jnp.int32) % (ACC_ROWS - N - 16)) + N + 16,
    ])
    edge_feat = jnp.zeros((EPAD, EF), jnp.float32).at[:E].set(edge_feat)
    edge_vector = jnp.zeros((EPAD, 3), jnp.float32).at[:E].set(
        edge_vector.astype(jnp.float32))

    xp = jnp.zeros((NPAD, H), jnp.float32).at[:N].set(x)
    xdp = jnp.zeros((NPAD, H), jnp.float32).at[:N].set(x_defect)
    vecp = jnp.zeros((NPAD, 3, H), jnp.float32).at[:N].set(vec)

    tj, ti = pl.pallas_call(
        _node_tables_body,
        grid=(NGRID,),
        in_specs=[pl.BlockSpec((NBLK, H), lambda g: (g, 0)),
                  pl.BlockSpec((NBLK, H), lambda g: (g, 0)),
                  pl.BlockSpec((NBLK, 3, H), lambda g: (g, 0, 0)),
                  pl.BlockSpec((H, 3 * H), lambda g: (0, 0)),
                  pl.BlockSpec((3 * H,), lambda g: (0,)),
                  pl.BlockSpec((H, H // 2), lambda g: (0, 0)),
                  pl.BlockSpec((H // 2,), lambda g: (0,)),
                  pl.BlockSpec((H // 2, 3 * H), lambda g: (0, 0)),
                  pl.BlockSpec((3 * H,), lambda g: (0,))],
        out_specs=(pl.BlockSpec((4, NBLK, 384), lambda g: (0, g, 0)),
                   pl.BlockSpec((4, NBLK, 128), lambda g: (0, g, 0))),
        out_shape=(jax.ShapeDtypeStruct((4, NPAD, 384), jnp.float32),
                   jax.ShapeDtypeStruct((4, NPAD, 128), jnp.float32)),
    )(xp, xdp, vecp, Wd, bd, W1, b1, W2, b2)

    rbf, jj, ii = pl.pallas_call(
        _rbf_body,
        grid=(EGRID,),
        in_specs=[pl.BlockSpec((EB, EF), lambda g: (g, 0)),
                  pl.BlockSpec((EF, 3 * H), lambda g: (0, 0)),
                  pl.BlockSpec((3 * H,), lambda g: (0,)),
                  pl.BlockSpec((1, 1, EB), lambda g: (g, 0, 0)),
                  pl.BlockSpec((1, 1, EB), lambda g: (g, 0, 0))],
        out_specs=(pl.BlockSpec((4, EB, 96), lambda g: (0, g, 0)),
                   pl.BlockSpec((4, EB), lambda g: (0, g)),
                   pl.BlockSpec((4, EB), lambda g: (0, g))),
        out_shape=(jax.ShapeDtypeStruct((4, EPAD, 96), jnp.float32),
                   jax.ShapeDtypeStruct((4, EPAD), jnp.int32),
                   jax.ShapeDtypeStruct((4, EPAD), jnp.int32)),
    )(edge_feat, We, be, jarr.reshape(EGRID, 1, EB), iarr.reshape(EGRID, 1, EB))

    tj2 = tj.reshape(4 * NPAD, 384)
    ti2 = ti.reshape(4 * NPAD, 128)
    ev = jnp.repeat(edge_vector * INV_SQRT_H, 16, axis=1)

    out_acc = _get_sc_edge()(tj2, ti2, rbf, jj, ii, iarr, ev)

    d_x, d_vec = pl.pallas_call(
        _act_body,
        grid=(NGRID,),
        in_specs=[pl.BlockSpec((4, NBLK, 128), lambda g: (0, g, 0)),
                  pl.BlockSpec((H, H), lambda g: (0, 0)),
                  pl.BlockSpec((H, 1), lambda g: (0, 0))],
        out_specs=(pl.BlockSpec((NBLK, H), lambda g: (g, 0)),
                   pl.BlockSpec((NBLK, 3, H), lambda g: (g, 0, 0))),
        out_shape=(jax.ShapeDtypeStruct((NPAD, H), jnp.float32),
                   jax.ShapeDtypeStruct((NPAD, 3, H), jnp.float32)),
    )(out_acc, Wv, Wg)

    return (d_x[:N], d_vec[:N])
